# baseline (speedup 1.0000x reference)
import functools

import jax
import jax.numpy as jnp
from jax import lax
from jax.experimental import pallas as pl
from jax.experimental.pallas import tpu as pltpu
from jax.experimental.pallas import tpu_sc as plsc

_NC = 2
_NS = 16
_NW = _NC * _NS
_L = 16

_CHUNK_B = 16


def _stab_body(f_ref, w_ref, b_ref, v_ref, o_ref):
    x = jnp.dot(f_ref[...], w_ref[...], preferred_element_type=jnp.float32)
    t = jnp.tanh(x + b_ref[...])
    o_ref[...] = jnp.dot(t, v_ref[...], preferred_element_type=jnp.float32)


def _score_table(features, W_att, b_att, v_ctx):
    n, d = features.shape
    att = W_att.shape[1]
    blk = 5000
    grid = n // blk
    out = pl.pallas_call(
        _stab_body,
        grid=(grid,),
        in_specs=[
            pl.BlockSpec((blk, d), lambda i: (i, 0)),
            pl.BlockSpec((d, att), lambda i: (0, 0)),
            pl.BlockSpec((1, att), lambda i: (0, 0)),
            pl.BlockSpec((att, 1), lambda i: (0, 0)),
        ],
        out_specs=pl.BlockSpec((blk, 1), lambda i: (i, 0)),
        out_shape=jax.ShapeDtypeStruct((n, 1), jnp.float32),
    )(features, W_att, b_att.reshape(1, att), v_ctx)
    return out.reshape(n)


def _make_sc_agg(n_nodes, d_feat, s_nbr, ct_total):
    S = s_nbr
    RPC = _CHUNK_B * S
    q, r = divmod(ct_total, _NW)
    CPW = q + 1
    assert CPW % 2 == 0, "2-deep ring needs an even per-worker chunk count"
    half = RPC // 2
    pad_chunks = _NW * CPW
    mesh = plsc.VectorSubcoreMesh(
        core_axis_name="c", subcore_axis_name="s",
        num_cores=_NC, num_subcores=_NS)

    @functools.partial(
        pl.kernel,
        out_type=jax.ShapeDtypeStruct((ct_total * _CHUNK_B, d_feat),
                                      jnp.float32),
        mesh=mesh,
        compiler_params=pltpu.CompilerParams(
            use_tc_tiling_on_sc=False, needs_layout_passes=False),
        scratch_types=[
            pltpu.VMEM((CPW * RPC,), jnp.int32),
            pltpu.VMEM((n_nodes,), jnp.float32),
            pltpu.VMEM((2, RPC, d_feat), jnp.float32),
            pltpu.VMEM((S, _L), jnp.float32),
            pltpu.VMEM((2, _CHUNK_B, d_feat), jnp.float32),
            pltpu.VMEM((_L,), jnp.float32),
            pltpu.SemaphoreType.DMA,
            pltpu.SemaphoreType.DMA,
            pltpu.SemaphoreType.DMA,
            pltpu.SemaphoreType.DMA,
        ],
    )
    def sc_agg(feat_hbm, stab_hbm, idxp_hbm, scale_hbm, out_hbm,
               idx_v, stab_v, rows_v, w_v, ob_v, sc_v,
               sem_r0, sem_r1, sem_o0, sem_o1):
        wid = lax.axis_index("s") * _NC + lax.axis_index("c")
        start = q * wid + jnp.minimum(wid, r)

        pltpu.sync_copy(stab_hbm, stab_v)
        pltpu.sync_copy(idxp_hbm.at[pl.ds(start * RPC, CPW * RPC)], idx_v)
        pltpu.sync_copy(scale_hbm, sc_v)
        sv = sc_v[...]
        sem_r = (sem_r0, sem_r1)
        sem_o = (sem_o0, sem_o1)

        def issue(k, p):
            ib = k * RPC
            pltpu.async_copy(
                feat_hbm.at[idx_v.at[pl.ds(ib, half)]],
                rows_v.at[p, pl.ds(0, half)], sem_r[p])
            pltpu.async_copy(
                feat_hbm.at[idx_v.at[pl.ds(ib + half, half)]],
                rows_v.at[p, pl.ds(half, half)], sem_r[p])

        def process(k, p, i):
            ibase = k * RPC
            pos0 = lax.iota(jnp.int32, _L) * S + ibase
            scores = []
            for s in range(S):
                iv = plsc.load_gather(idx_v, [pos0 + s])
                scores.append(plsc.load_gather(stab_v, [iv]))
            m = scores[0]
            for s in range(1, S):
                m = jnp.maximum(m, scores[s])
            exps = [jnp.exp(x - m) for x in scores]
            tot = exps[0]
            for s in range(1, S):
                tot = tot + exps[s]
            wfac = sv / tot
            for s in range(S):
                w_v[s] = exps[s] * wfac

            pltpu.make_async_copy(
                feat_hbm.at[pl.ds(0, RPC)], rows_v.at[p], sem_r[p]).wait()

            @pl.when(i >= 2)
            def _():
                pltpu.make_async_copy(
                    feat_hbm.at[pl.ds(0, _CHUNK_B)], ob_v.at[p],
                    sem_o[p]).wait()

            def b_body(b, _):
                bidx = jnp.full((_L,), b, jnp.int32)
                wb = [
                    plsc.load_gather(
                        w_v, [jnp.full((_L,), s, jnp.int32), bidx])
                    for s in range(S)
                ]
                for kk in range(d_feat // _L):
                    ks = pl.ds(kk * _L, _L)
                    acc = wb[0] * rows_v[p, b * S, ks]
                    for s in range(1, S):
                        acc = acc + wb[s] * rows_v[p, b * S + s, ks]
                    ob_v[p, b, ks] = jnp.maximum(acc, 0.0)
                return _

            lax.fori_loop(0, _CHUNK_B, b_body, None)

            @pl.when(start + k < ct_total)
            def _():
                row0 = (start + k) * _CHUNK_B
                pltpu.async_copy(
                    ob_v.at[p], out_hbm.at[pl.ds(row0, _CHUNK_B)], sem_o[p])

        issue(0, 0)

        def pair_body(i, _):
            k0 = 2 * i
            issue(k0 + 1, 1)
            process(k0, 0, k0)

            @pl.when(i < CPW // 2 - 1)
            def _():
                issue(k0 + 2, 0)

            process(k0 + 1, 1, k0 + 1)
            return _

        lax.fori_loop(0, CPW // 2, pair_body, None)

        for p in range(2):
            @pl.when(start + (CPW - 2 + p) < ct_total)
            def _():
                pltpu.make_async_copy(
                    feat_hbm.at[pl.ds(0, _CHUNK_B)], ob_v.at[p],
                    sem_o[p]).wait()

    return sc_agg, CPW, pad_chunks


def kernel(features, nodes, neigh_idx, W_att, b_att, v_ctx, num_sample):
    del nodes
    n_nodes, d_feat = features.shape
    b_sz, s_nbr = neigh_idx.shape

    b16 = ((b_sz + _CHUNK_B - 1) // _CHUNK_B) * _CHUNK_B
    ct_total = b16 // _CHUNK_B

    sc_agg, cpw, pad_chunks = _make_sc_agg(n_nodes, d_feat, s_nbr, ct_total)

    idxp = jnp.pad(neigh_idx.reshape(-1),
                   (0, (pad_chunks * _CHUNK_B - b_sz) * s_nbr))

    stab = _score_table(features, W_att, b_att, v_ctx)
    scale = jnp.full((_L,), 1.0, jnp.float32) / num_sample

    out = sc_agg(features, stab, idxp, scale)
    return out[:b_sz]


# speedup vs baseline: 1.3493x; 1.3493x over previous
import functools

import jax
import jax.numpy as jnp
from jax import lax
from jax.experimental import pallas as pl
from jax.experimental.pallas import tpu as pltpu
from jax.experimental.pallas import tpu_sc as plsc

_NC = 2
_NS = 16
_NW = _NC * _NS
_L = 16

_CHUNK_B = 16


def _stab_body(f_ref, w_ref, b_ref, v_ref, o_ref):
    x = jnp.dot(f_ref[...], w_ref[...], preferred_element_type=jnp.float32)
    t = jnp.tanh(x + b_ref[...])
    o_ref[...] = jnp.dot(t, v_ref[...], preferred_element_type=jnp.float32)


def _score_table(features, W_att, b_att, v_ctx):
    n, d = features.shape
    att = W_att.shape[1]
    blk = 5000
    grid = n // blk
    out = pl.pallas_call(
        _stab_body,
        grid=(grid,),
        in_specs=[
            pl.BlockSpec((blk, d), lambda i: (i, 0)),
            pl.BlockSpec((d, att), lambda i: (0, 0)),
            pl.BlockSpec((1, att), lambda i: (0, 0)),
            pl.BlockSpec((att, 1), lambda i: (0, 0)),
        ],
        out_specs=pl.BlockSpec((blk, 1), lambda i: (i, 0)),
        out_shape=jax.ShapeDtypeStruct((n, 1), jnp.float32),
    )(features, W_att, b_att.reshape(1, att), v_ctx)
    return out.reshape(n)


def _make_sc_agg(n_nodes, d_feat, s_nbr, ct_total):
    S = s_nbr
    RPC = _CHUNK_B * S
    q, r = divmod(ct_total, _NW)
    CPW = q + 1
    assert CPW % 2 == 0, "2-deep ring needs an even per-worker chunk count"
    half = RPC // 2
    mesh = plsc.VectorSubcoreMesh(
        core_axis_name="c", subcore_axis_name="s",
        num_cores=_NC, num_subcores=_NS)

    @functools.partial(
        pl.kernel,
        out_type=jax.ShapeDtypeStruct((ct_total * _CHUNK_B, d_feat),
                                      jnp.float32),
        mesh=mesh,
        compiler_params=pltpu.CompilerParams(
            use_tc_tiling_on_sc=False, needs_layout_passes=False),
        scratch_types=[
            pltpu.VMEM((CPW * RPC,), jnp.int32),
            pltpu.VMEM((n_nodes,), jnp.float32),
            pltpu.VMEM((2, RPC, d_feat), jnp.float32),
            pltpu.VMEM((S, _L), jnp.float32),
            pltpu.VMEM((2, _CHUNK_B, d_feat), jnp.float32),
            pltpu.VMEM((_L,), jnp.float32),
            pltpu.SemaphoreType.DMA,
            pltpu.SemaphoreType.DMA,
            pltpu.SemaphoreType.DMA,
            pltpu.SemaphoreType.DMA,
        ],
    )
    def sc_agg(feat_hbm, nidx_hbm, stab_hbm, scale_hbm, out_hbm,
               idx_v, stab_v, rows_v, w_v, ob_v, sc_v,
               sem_r0, sem_r1, sem_o0, sem_o1):
        wid = lax.axis_index("s") * _NC + lax.axis_index("c")
        start = q * wid + jnp.minimum(wid, r)

        pltpu.sync_copy(stab_hbm, stab_v)

        @pl.when(start + CPW <= ct_total)
        def _():
            pltpu.sync_copy(
                nidx_hbm.at[pl.ds(start * RPC, CPW * RPC)], idx_v)

        @pl.when(start + CPW > ct_total)
        def _():
            pltpu.sync_copy(
                nidx_hbm.at[pl.ds(start * RPC, (CPW - 1) * RPC)],
                idx_v.at[pl.ds(0, (CPW - 1) * RPC)])
            zero = jnp.zeros((_L,), jnp.int32)
            for j in range(RPC // _L):
                idx_v[pl.ds((CPW - 1) * RPC + j * _L, _L)] = zero

        pltpu.sync_copy(scale_hbm, sc_v)
        sv = sc_v[...]
        sem_r = (sem_r0, sem_r1)
        sem_o = (sem_o0, sem_o1)

        def chunk_indices(k):
            pos0 = lax.iota(jnp.int32, _L) * S + k * RPC
            return [plsc.load_gather(idx_v, [pos0 + s]) for s in range(S)]

        def issue(k, p):
            ib = k * RPC
            pltpu.async_copy(
                feat_hbm.at[idx_v.at[pl.ds(ib, half)]],
                rows_v.at[p, pl.ds(0, half)], sem_r[p])
            pltpu.async_copy(
                feat_hbm.at[idx_v.at[pl.ds(ib + half, half)]],
                rows_v.at[p, pl.ds(half, half)], sem_r[p])

        def process(k, p):
            scores = [plsc.load_gather(stab_v, [iv])
                      for iv in chunk_indices(k)]
            m = scores[0]
            for s in range(1, S):
                m = jnp.maximum(m, scores[s])
            exps = [jnp.exp(x - m) for x in scores]
            tot = exps[0]
            for s in range(1, S):
                tot = tot + exps[s]
            wfac = sv / tot
            for s in range(S):
                w_v[s] = exps[s] * wfac

            pltpu.make_async_copy(
                feat_hbm.at[pl.ds(0, RPC)], rows_v.at[p], sem_r[p]).wait()

            @pl.when(k >= 2)
            def _():
                pltpu.make_async_copy(
                    feat_hbm.at[pl.ds(0, _CHUNK_B)], ob_v.at[p],
                    sem_o[p]).wait()

            @plsc.parallel_loop(0, _CHUNK_B)
            def b_body(b):
                bidx = jnp.full((_L,), b, jnp.int32)
                wb = [
                    plsc.load_gather(
                        w_v, [jnp.full((_L,), s, jnp.int32), bidx])
                    for s in range(S)
                ]
                for kk in range(d_feat // _L):
                    ks = pl.ds(kk * _L, _L)
                    terms = [wb[s] * rows_v[p, b * S + s, ks]
                             for s in range(S)]
                    while len(terms) > 1:
                        nxt = [terms[j] + terms[j + 1]
                               for j in range(0, len(terms) - 1, 2)]
                        if len(terms) % 2:
                            nxt.append(terms[-1])
                        terms = nxt
                    ob_v[p, b, ks] = jnp.maximum(terms[0], 0.0)

            @pl.when(start + k < ct_total)
            def _():
                row0 = (start + k) * _CHUNK_B
                pltpu.async_copy(
                    ob_v.at[p], out_hbm.at[pl.ds(row0, _CHUNK_B)], sem_o[p])

        issue(0, 0)

        def pair_body(i, _):
            k0 = 2 * i
            issue(k0 + 1, 1)
            process(k0, 0)

            @pl.when(i < CPW // 2 - 1)
            def _():
                issue(k0 + 2, 0)

            process(k0 + 1, 1)
            return _

        lax.fori_loop(0, CPW // 2, pair_body, None)

        for p in range(2):
            @pl.when(start + (CPW - 2 + p) < ct_total)
            def _():
                pltpu.make_async_copy(
                    feat_hbm.at[pl.ds(0, _CHUNK_B)], ob_v.at[p],
                    sem_o[p]).wait()

    return sc_agg


def kernel(features, nodes, neigh_idx, W_att, b_att, v_ctx, num_sample):
    del nodes
    n_nodes, d_feat = features.shape
    b_sz, s_nbr = neigh_idx.shape
    assert b_sz % _CHUNK_B == 0
    ct_total = b_sz // _CHUNK_B

    sc_agg = _make_sc_agg(n_nodes, d_feat, s_nbr, ct_total)

    stab = _score_table(features, W_att, b_att, v_ctx)
    scale = jnp.full((_L,), 1.0, jnp.float32) / num_sample

    return sc_agg(features, neigh_idx.reshape(-1), stab, scale)
